# baseline (speedup 1.0000x reference)
import functools

import jax
import jax.numpy as jnp
from jax import lax
from jax.experimental import pallas as pl
from jax.experimental.pallas import tpu as pltpu
from jax.experimental.pallas import tpu_sc as plsc

_NUM_ITEMS = 1000000
_D = 64
_BATCH = 16384
_HIST = 50

_NC = 2
_NS = 16
_NW = _NC * _NS
_BPW = _BATCH // _NW

_LC = 512
_NBLK = pl.cdiv(_NUM_ITEMS, _LC)
_NJ = _NBLK // 2
_H = _NJ * _LC


def _linearize_body(lo_ref, hi_ref, out_ref):
    out_ref[...] = jnp.concatenate([lo_ref[...].T, hi_ref[...].T], axis=1)


def _gather_body(xt_hbm, table_hbm, out_hbm, idx0, idx1, rows0, rows1, tbuf,
                 isem0, isem1, gsem0, gsem1, osem):
    wid = lax.axis_index("s") * _NC + lax.axis_index("c")
    b0 = wid * _BPW

    idxb = (idx0, idx1)
    rows = (rows0, rows1)
    isem = (isem0, isem1)
    gsem = (gsem0, gsem1)

    def fetch_idx(h, s):
        pltpu.async_copy(xt_hbm.at[h, pl.ds(b0, _BPW)], idxb[s], isem[s])

    def start_gather(h, s):
        pltpu.make_async_copy(
            xt_hbm.at[h, pl.ds(b0, _BPW)], idxb[s], isem[s]
        ).wait()
        @pl.loop(0, _BPW, step=16)
        def _(g):
            v = idxb[s][pl.ds(g, 16)]
            m = jnp.where(v >= _H, jnp.int32(1), jnp.int32(0))
            idxb[s][pl.ds(g, 16)] = 2 * (v - m * _H) + m

        pltpu.async_copy(table_hbm.at[idxb[s]], rows[s], gsem[s])

    def wait_gather(s):
        pltpu.make_async_copy(table_hbm.at[idxb[s]], rows[s], gsem[s]).wait()

    def wait_store(h):
        pltpu.make_async_copy(
            tbuf, out_hbm.at[h, :, pl.ds(b0, _BPW)], osem
        ).wait()

    fetch_idx(0, 0)
    fetch_idx(1, 1)
    start_gather(0, 0)

    @pl.loop(0, _HIST, step=2)
    def _(h):
        for s in range(2):
            cur = h + s
            wait_gather(s)

            @pl.when(cur + 2 < _HIST)
            def _():
                fetch_idx(cur + 2, s)

            @pl.when(cur + 1 < _HIST)
            def _():
                start_gather(cur + 1, 1 - s)

            @pl.when(cur >= 1)
            def _():
                wait_store(cur - 1)

            @pl.loop(0, _D)
            def _(d):
                dvec = jnp.full((16,), d, jnp.int32)
                for g in range(_BPW // 16):
                    v = plsc.load_gather(
                        rows[s], [g * 16 + lax.iota(jnp.int32, 16), dvec]
                    )
                    tbuf[d, pl.ds(g * 16, 16)] = v

            pltpu.async_copy(tbuf, out_hbm.at[cur, :, pl.ds(b0, _BPW)], osem)

    wait_store(_HIST - 1)


@jax.jit
def _embedding_lookup(x_t, table_t):
    lin = pl.pallas_call(
        _linearize_body,
        grid=(_NJ,),
        in_specs=[
            pl.BlockSpec((_D, _LC), lambda j: (0, j)),
            pl.BlockSpec((_D, _LC), lambda j: (0, j + _NJ)),
        ],
        out_specs=pl.BlockSpec((_LC, 2 * _D), lambda j: (j, 0)),
        out_shape=jax.ShapeDtypeStruct((_H, 2 * _D), jnp.float32),
    )(table_t, table_t)
    lin = lin.reshape(2 * _H, _D)

    mesh = plsc.VectorSubcoreMesh(core_axis_name="c", subcore_axis_name="s")
    call = functools.partial(
        pl.kernel,
        mesh=mesh,
        out_type=jax.ShapeDtypeStruct((_HIST, _D, _BATCH), jnp.float32),
        scratch_types=(
            [pltpu.VMEM((_BPW,), jnp.int32) for _ in range(2)]
            + [pltpu.VMEM((_BPW, _D), jnp.float32) for _ in range(2)]
            + [pltpu.VMEM((_D, _BPW), jnp.float32)]
            + [pltpu.SemaphoreType.DMA for _ in range(5)]
        ),
        compiler_params=pltpu.CompilerParams(
            use_tc_tiling_on_sc=False, needs_layout_passes=False
        ),
    )(_gather_body)
    return call(x_t, lin)


def kernel(x, embedding_u):
    x_t = jnp.transpose(x).astype(jnp.int32)
    table_t = jnp.transpose(embedding_u)
    out_phys = _embedding_lookup(x_t, table_t)
    return jnp.transpose(out_phys, (2, 0, 1))


# speedup vs baseline: 1.8888x; 1.8888x over previous
import functools

import jax
import jax.numpy as jnp
from jax import lax
from jax.experimental import pallas as pl
from jax.experimental.pallas import tpu as pltpu
from jax.experimental.pallas import tpu_sc as plsc

_NUM_ITEMS = 1000000
_D = 64
_BATCH = 16384
_HIST = 50

_NC = 2
_NS = 16
_NW = _NC * _NS
_BPW = _BATCH // _NW

_LC = 2048
_NJ = pl.cdiv(_NUM_ITEMS // 2, _LC)
_H = _NJ * _LC


def _linearize_body(lo_ref, hi_ref, out_ref):
    out_ref[...] = jnp.concatenate([lo_ref[...].T, hi_ref[...].T], axis=1)


def _gather_body(xt_hbm, table_hbm, out_hbm, idx0, idx1, rows0, rows1, tbuf,
                 isem0, isem1, gsem0, gsem1, osem):
    wid = lax.axis_index("s") * _NC + lax.axis_index("c")
    b0 = wid * _BPW

    idxb = (idx0, idx1)
    rows = (rows0, rows1)
    isem = (isem0, isem1)
    gsem = (gsem0, gsem1)

    def fetch_idx(h, s):
        pltpu.async_copy(xt_hbm.at[h, pl.ds(b0, _BPW)], idxb[s], isem[s])

    def start_gather(h, s):
        pltpu.make_async_copy(
            xt_hbm.at[h, pl.ds(b0, _BPW)], idxb[s], isem[s]
        ).wait()
        @pl.loop(0, _BPW, step=16)
        def _(g):
            v = idxb[s][pl.ds(g, 16)]
            m = jnp.where(v >= _H, jnp.int32(1), jnp.int32(0))
            idxb[s][pl.ds(g, 16)] = 2 * (v - m * _H) + m

        pltpu.async_copy(table_hbm.at[idxb[s]], rows[s], gsem[s])

    def wait_gather(s):
        pltpu.make_async_copy(table_hbm.at[idxb[s]], rows[s], gsem[s]).wait()

    def wait_store(h):
        pltpu.make_async_copy(
            tbuf, out_hbm.at[h, :, pl.ds(b0, _BPW)], osem
        ).wait()

    fetch_idx(0, 0)
    fetch_idx(1, 1)
    start_gather(0, 0)

    @pl.loop(0, _HIST, step=2)
    def _(h):
        for s in range(2):
            cur = h + s
            wait_gather(s)

            @pl.when(cur + 2 < _HIST)
            def _():
                fetch_idx(cur + 2, s)

            @pl.when(cur + 1 < _HIST)
            def _():
                start_gather(cur + 1, 1 - s)

            @pl.when(cur >= 1)
            def _():
                wait_store(cur - 1)

            @pl.loop(0, _BPW // 16)
            def _(g):
                gv = g * 16 + lax.iota(jnp.int32, 16)
                for d in range(_D):
                    dv = (d + lax.iota(jnp.int32, 16)) & (_D - 1)
                    v = plsc.load_gather(rows[s], [gv, dv])
                    plsc.store_scatter(tbuf, [dv, gv], v)

            pltpu.async_copy(tbuf, out_hbm.at[cur, :, pl.ds(b0, _BPW)], osem)

    wait_store(_HIST - 1)


@jax.jit
def _embedding_lookup(x_t, table_t):
    lin = pl.pallas_call(
        _linearize_body,
        grid=(_NJ,),
        in_specs=[
            pl.BlockSpec((_D, _LC), lambda j: (0, j)),
            pl.BlockSpec(
                (_D, _LC),
                lambda j: (0, jnp.where(j + _NJ < pl.cdiv(_NUM_ITEMS, _LC),
                                        j + _NJ, 0)),
            ),
        ],
        out_specs=pl.BlockSpec((_LC, 2 * _D), lambda j: (j, 0)),
        out_shape=jax.ShapeDtypeStruct((_H, 2 * _D), jnp.float32),
    )(table_t, table_t)
    lin = lin.reshape(2 * _H, _D)

    mesh = plsc.VectorSubcoreMesh(core_axis_name="c", subcore_axis_name="s")
    call = functools.partial(
        pl.kernel,
        mesh=mesh,
        out_type=jax.ShapeDtypeStruct((_HIST, _D, _BATCH), jnp.float32),
        scratch_types=(
            [pltpu.VMEM((_BPW,), jnp.int32) for _ in range(2)]
            + [pltpu.VMEM((_BPW, _D), jnp.float32) for _ in range(2)]
            + [pltpu.VMEM((_D, _BPW), jnp.float32)]
            + [pltpu.SemaphoreType.DMA for _ in range(5)]
        ),
        compiler_params=pltpu.CompilerParams(
            use_tc_tiling_on_sc=False, needs_layout_passes=False
        ),
    )(_gather_body)
    return call(x_t, lin)


def kernel(x, embedding_u):
    x_t = jnp.transpose(x).astype(jnp.int32)
    table_t = jnp.transpose(embedding_u)
    out_phys = _embedding_lookup(x_t, table_t)
    return jnp.transpose(out_phys, (2, 0, 1))


# speedup vs baseline: 2.3394x; 1.2386x over previous
import functools

import jax
import jax.numpy as jnp
from jax import lax
from jax.experimental import pallas as pl
from jax.experimental.pallas import tpu as pltpu
from jax.experimental.pallas import tpu_sc as plsc

_NUM_ITEMS = 1000000
_D = 64
_BATCH = 16384
_HIST = 50

_NC = 2
_NS = 16
_NW = _NC * _NS
_BPW = _BATCH // _NW

_LC = 4096
_NJ = pl.cdiv(_NUM_ITEMS // 2, _LC)
_H = _NJ * _LC


def _linearize_body(lo_ref, hi_ref, out_ref):
    out_ref[...] = jnp.concatenate([lo_ref[...].T, hi_ref[...].T], axis=1)


def _gather_body(xt_hbm, table_hbm, out_hbm, idx0, idx1, rows0, rows1, tbuf,
                 isem0, isem1, gsem0, gsem1, osem):
    wid = lax.axis_index("s") * _NC + lax.axis_index("c")
    b0 = wid * _BPW

    idxb = (idx0, idx1)
    rows = (rows0, rows1)
    isem = (isem0, isem1)
    gsem = (gsem0, gsem1)

    def fetch_idx(h, s):
        pltpu.async_copy(xt_hbm.at[h, pl.ds(b0, _BPW)], idxb[s], isem[s])

    def start_gather(h, s):
        pltpu.make_async_copy(
            xt_hbm.at[h, pl.ds(b0, _BPW)], idxb[s], isem[s]
        ).wait()
        @pl.loop(0, _BPW, step=16)
        def _(g):
            v = idxb[s][pl.ds(g, 16)]
            m = jnp.where(v >= _H, jnp.int32(1), jnp.int32(0))
            idxb[s][pl.ds(g, 16)] = 2 * (v - m * _H) + m

        pltpu.async_copy(table_hbm.at[idxb[s]], rows[s], gsem[s])

    def wait_gather(s):
        pltpu.make_async_copy(table_hbm.at[idxb[s]], rows[s], gsem[s]).wait()

    def wait_store(h):
        pltpu.make_async_copy(
            tbuf, out_hbm.at[h, :, pl.ds(b0, _BPW)], osem
        ).wait()

    fetch_idx(0, 0)
    fetch_idx(1, 1)
    start_gather(0, 0)

    @pl.loop(0, _HIST, step=2)
    def _(h):
        for s in range(2):
            cur = h + s
            wait_gather(s)

            @pl.when(cur + 2 < _HIST)
            def _():
                fetch_idx(cur + 2, s)

            @pl.when(cur + 1 < _HIST)
            def _():
                start_gather(cur + 1, 1 - s)

            @pl.when(cur >= 1)
            def _():
                wait_store(cur - 1)

            @pl.loop(0, _D)
            def _(d):
                dv = (d + lax.iota(jnp.int32, 16)) & (_D - 1)
                iv = lax.iota(jnp.int32, 16)
                for g in range(_BPW // 16):
                    gv = iv + g * 16
                    v = plsc.load_gather(rows[s], [gv, dv])
                    plsc.store_scatter(tbuf, [dv, gv], v)

            pltpu.async_copy(tbuf, out_hbm.at[cur, :, pl.ds(b0, _BPW)], osem)

    wait_store(_HIST - 1)


@jax.jit
def _embedding_lookup(x_t, table_t):
    lin = pl.pallas_call(
        _linearize_body,
        grid=(_NJ,),
        in_specs=[
            pl.BlockSpec((_D, _LC), lambda j: (0, j)),
            pl.BlockSpec(
                (_D, _LC),
                lambda j: (0, jnp.where(j + _NJ < pl.cdiv(_NUM_ITEMS, _LC),
                                        j + _NJ, 0)),
            ),
        ],
        out_specs=pl.BlockSpec((_LC, 2 * _D), lambda j: (j, 0)),
        out_shape=jax.ShapeDtypeStruct((_H, 2 * _D), jnp.float32),
    )(table_t, table_t)
    lin = lin.reshape(2 * _H, _D)

    mesh = plsc.VectorSubcoreMesh(core_axis_name="c", subcore_axis_name="s")
    call = functools.partial(
        pl.kernel,
        mesh=mesh,
        out_type=jax.ShapeDtypeStruct((_HIST, _D, _BATCH), jnp.float32),
        scratch_types=(
            [pltpu.VMEM((_BPW,), jnp.int32) for _ in range(2)]
            + [pltpu.VMEM((_BPW, _D), jnp.float32) for _ in range(2)]
            + [pltpu.VMEM((_D, _BPW), jnp.float32)]
            + [pltpu.SemaphoreType.DMA for _ in range(5)]
        ),
        compiler_params=pltpu.CompilerParams(
            use_tc_tiling_on_sc=False, needs_layout_passes=False
        ),
    )(_gather_body)
    return call(x_t, lin)


def kernel(x, embedding_u):
    x_t = jnp.transpose(x).astype(jnp.int32)
    table_t = jnp.transpose(embedding_u)
    out_phys = _embedding_lookup(x_t, table_t)
    return jnp.transpose(out_phys, (2, 0, 1))


# speedup vs baseline: 2.9504x; 1.2612x over previous
import functools

import jax
import jax.numpy as jnp
from jax import lax
from jax.experimental import pallas as pl
from jax.experimental.pallas import tpu as pltpu
from jax.experimental.pallas import tpu_sc as plsc

_NUM_ITEMS = 1000000
_D = 64
_BATCH = 16384
_HIST = 50

_NC = 2
_NS = 16
_NW = _NC * _NS
_BPW = _BATCH // _NW

_LC = 4096
_NJ = pl.cdiv(_NUM_ITEMS // 2, _LC)
_H = _NJ * _LC


def _linearize_body(lo_ref, hi_ref, out_ref):
    out_ref[...] = jnp.concatenate([lo_ref[...].T, hi_ref[...].T], axis=1)


def _gather_body(xt_hbm, table_hbm, out_hbm, idx0, idx1, rows0, rows1, tbuf,
                 isem0, isem1, gsem0, gsem1, osem):
    wid = lax.axis_index("s") * _NC + lax.axis_index("c")
    b0 = wid * _BPW

    idxb = (idx0, idx1)
    rows = (rows0, rows1)
    isem = (isem0, isem1)
    gsem = (gsem0, gsem1)

    def fetch_idx(h, s):
        pltpu.async_copy(xt_hbm.at[h, pl.ds(b0, _BPW)], idxb[s], isem[s])

    def start_gather(h, s):
        pltpu.make_async_copy(
            xt_hbm.at[h, pl.ds(b0, _BPW)], idxb[s], isem[s]
        ).wait()
        @pl.loop(0, _BPW, step=16)
        def _(g):
            v = idxb[s][pl.ds(g, 16)]
            m = jnp.where(v >= _H, jnp.int32(1), jnp.int32(0))
            idxb[s][pl.ds(g, 16)] = 2 * (v - m * _H) + m

        pltpu.async_copy(table_hbm.at[idxb[s]], rows[s], gsem[s])

    def wait_gather(s):
        pltpu.make_async_copy(table_hbm.at[idxb[s]], rows[s], gsem[s]).wait()

    c0 = wid * (_BPW // 128)

    def wait_store(h):
        pltpu.make_async_copy(
            tbuf, out_hbm.at[h, :, pl.ds(c0, _BPW // 128), :], osem
        ).wait()

    fetch_idx(0, 0)
    fetch_idx(1, 1)
    start_gather(0, 0)

    @pl.loop(0, _HIST, step=2)
    def _(h):
        for s in range(2):
            cur = h + s
            wait_gather(s)

            @pl.when(cur + 2 < _HIST)
            def _():
                fetch_idx(cur + 2, s)

            @pl.when(cur + 1 < _HIST)
            def _():
                start_gather(cur + 1, 1 - s)

            @pl.when(cur >= 1)
            def _():
                wait_store(cur - 1)

            @pl.loop(0, _D)
            def _(d):
                iv = lax.iota(jnp.int32, 16)
                dv = (d + iv) & (_D - 1)
                rv = dv >> 3
                wb = (dv & 7) * 128 + iv
                for g in range(_BPW // 16):
                    gv = iv + g * 16
                    v = plsc.load_gather(rows[s], [gv, dv])
                    cv = jnp.full((16,), g >> 3, jnp.int32)
                    wv = wb + (g & 7) * 16
                    plsc.store_scatter(tbuf, [rv, cv, wv], v)

            pltpu.async_copy(
                tbuf, out_hbm.at[cur, :, pl.ds(c0, _BPW // 128), :], osem
            )

    wait_store(_HIST - 1)


@jax.jit
def _embedding_lookup(x_t, table_t):
    lin = pl.pallas_call(
        _linearize_body,
        grid=(_NJ,),
        in_specs=[
            pl.BlockSpec((_D, _LC), lambda j: (0, j)),
            pl.BlockSpec(
                (_D, _LC),
                lambda j: (0, jnp.where(j + _NJ < pl.cdiv(_NUM_ITEMS, _LC),
                                        j + _NJ, 0)),
            ),
        ],
        out_specs=pl.BlockSpec((_LC, 2 * _D), lambda j: (j, 0)),
        out_shape=jax.ShapeDtypeStruct((_H, 2 * _D), jnp.float32),
    )(table_t, table_t)
    lin = lin.reshape(2 * _H, _D)

    mesh = plsc.VectorSubcoreMesh(core_axis_name="c", subcore_axis_name="s")
    call = functools.partial(
        pl.kernel,
        mesh=mesh,
        out_type=jax.ShapeDtypeStruct(
            (_HIST, _D // 8, _BATCH // 128, 1024), jnp.float32
        ),
        scratch_types=(
            [pltpu.VMEM((_BPW,), jnp.int32) for _ in range(2)]
            + [pltpu.VMEM((_BPW, _D), jnp.float32) for _ in range(2)]
            + [pltpu.VMEM((_D // 8, _BPW // 128, 1024), jnp.float32)]
            + [pltpu.SemaphoreType.DMA for _ in range(5)]
        ),
        compiler_params=pltpu.CompilerParams(
            use_tc_tiling_on_sc=False, needs_layout_passes=False
        ),
    )(_gather_body)
    return call(x_t, lin)


def kernel(x, embedding_u):
    x_t = jnp.transpose(x).astype(jnp.int32)
    table_t = jnp.transpose(embedding_u)
    out4 = _embedding_lookup(x_t, table_t)
    r5 = out4.reshape(_HIST, 8, _BATCH // 128, 8, 128)
    t5 = r5.transpose(2, 4, 0, 1, 3)
    return t5.reshape(_BATCH, _HIST, _D)


# speedup vs baseline: 4.3660x; 1.4798x over previous
import functools

import jax
import jax.numpy as jnp
from jax import lax
from jax.experimental import pallas as pl
from jax.experimental.pallas import tpu as pltpu
from jax.experimental.pallas import tpu_sc as plsc

_NUM_ITEMS = 1000000
_D = 64
_BATCH = 16384
_HIST = 50

_NC = 2
_NS = 16
_NW = _NC * _NS
_BPW = _BATCH // _NW

_LC = 4096
_NJ = pl.cdiv(_NUM_ITEMS // 2, _LC)
_H = _NJ * _LC


def _linearize_body(lo_ref, hi_ref, out_ref):
    out_ref[...] = jnp.concatenate([lo_ref[...].T, hi_ref[...].T], axis=1)


def _gather_body(xt_hbm, table_hbm, out_hbm, idx0, idx1, rows0, rows1, tbuf,
                 isem0, isem1, gsem0, gsem1, osem):
    wid = lax.axis_index("s") * _NC + lax.axis_index("c")
    b0 = wid * _BPW

    idxb = (idx0, idx1)
    rows = (rows0, rows1)
    isem = (isem0, isem1)
    gsem = (gsem0, gsem1)

    def fetch_idx(h, s):
        pltpu.async_copy(xt_hbm.at[h, pl.ds(b0, _BPW)], idxb[s], isem[s])

    def start_gather(h, s):
        pltpu.make_async_copy(
            xt_hbm.at[h, pl.ds(b0, _BPW)], idxb[s], isem[s]
        ).wait()
        @pl.loop(0, _BPW, step=16)
        def _(g):
            v = idxb[s][pl.ds(g, 16)]
            m = jnp.where(v >= _H, jnp.int32(1), jnp.int32(0))
            idxb[s][pl.ds(g, 16)] = 2 * (v - m * _H) + m

        pltpu.async_copy(table_hbm.at[idxb[s]], rows[s], gsem[s])

    def wait_gather(s):
        pltpu.make_async_copy(table_hbm.at[idxb[s]], rows[s], gsem[s]).wait()

    c0 = wid * (_BPW // 128)

    def wait_store(h):
        pltpu.make_async_copy(
            tbuf, out_hbm.at[h, :, pl.ds(c0, _BPW // 128), :], osem
        ).wait()

    fetch_idx(0, 0)
    fetch_idx(1, 1)
    start_gather(0, 0)

    @pl.loop(0, _HIST, step=2)
    def _(h):
        for s in range(2):
            cur = h + s
            wait_gather(s)

            @pl.when(cur + 2 < _HIST)
            def _():
                fetch_idx(cur + 2, s)

            @pl.when(cur + 1 < _HIST)
            def _():
                start_gather(cur + 1, 1 - s)

            @pl.when(cur >= 1)
            def _():
                wait_store(cur - 1)

            @pl.loop(0, _D)
            def _(d):
                iv = lax.iota(jnp.int32, 16)
                dv = (d + iv) & (_D - 1)
                rv = dv >> 3
                wb = (dv & 7) * 128 + iv
                for g0 in range(0, _BPW // 16, 8):
                    vs = [
                        plsc.load_gather(rows[s], [iv + (g0 + k) * 16, dv])
                        for k in range(8)
                    ]
                    for k in range(8):
                        g = g0 + k
                        cv = jnp.full((16,), g >> 3, jnp.int32)
                        wv = wb + (g & 7) * 16
                        plsc.store_scatter(tbuf, [rv, cv, wv], vs[k])

            pltpu.async_copy(
                tbuf, out_hbm.at[cur, :, pl.ds(c0, _BPW // 128), :], osem
            )

    wait_store(_HIST - 1)


@jax.jit
def _embedding_lookup(x_t, table_t):
    lin = pl.pallas_call(
        _linearize_body,
        grid=(_NJ,),
        in_specs=[
            pl.BlockSpec((_D, _LC), lambda j: (0, j)),
            pl.BlockSpec(
                (_D, _LC),
                lambda j: (0, jnp.where(j + _NJ < pl.cdiv(_NUM_ITEMS, _LC),
                                        j + _NJ, 0)),
            ),
        ],
        out_specs=pl.BlockSpec((_LC, 2 * _D), lambda j: (j, 0)),
        out_shape=jax.ShapeDtypeStruct((_H, 2 * _D), jnp.float32),
    )(table_t, table_t)
    lin = lin.reshape(2 * _H, _D)

    mesh = plsc.VectorSubcoreMesh(core_axis_name="c", subcore_axis_name="s")
    call = functools.partial(
        pl.kernel,
        mesh=mesh,
        out_type=jax.ShapeDtypeStruct(
            (_HIST, _D // 8, _BATCH // 128, 1024), jnp.float32
        ),
        scratch_types=(
            [pltpu.VMEM((_BPW,), jnp.int32) for _ in range(2)]
            + [pltpu.VMEM((_BPW, _D), jnp.float32) for _ in range(2)]
            + [pltpu.VMEM((_D // 8, _BPW // 128, 1024), jnp.float32)]
            + [pltpu.SemaphoreType.DMA for _ in range(5)]
        ),
        compiler_params=pltpu.CompilerParams(
            use_tc_tiling_on_sc=False, needs_layout_passes=False
        ),
    )(_gather_body)
    return call(x_t, lin)


def kernel(x, embedding_u):
    x_t = jnp.transpose(x).astype(jnp.int32)
    table_t = jnp.transpose(embedding_u)
    out4 = _embedding_lookup(x_t, table_t)
    r5 = out4.reshape(_HIST, 8, _BATCH // 128, 8, 128)
    t5 = r5.transpose(2, 4, 0, 1, 3)
    return t5.reshape(_BATCH, _HIST, _D)


# speedup vs baseline: 4.6532x; 1.0658x over previous
import functools

import jax
import jax.numpy as jnp
from jax import lax
from jax.experimental import pallas as pl
from jax.experimental.pallas import tpu as pltpu
from jax.experimental.pallas import tpu_sc as plsc

_NUM_ITEMS = 1000000
_D = 64
_BATCH = 16384
_HIST = 50

_NC = 2
_NS = 16
_NW = _NC * _NS
_BPW = _BATCH // _NW

_LC = 8192
_NJ = pl.cdiv(_NUM_ITEMS // 2, _LC)
_H = _NJ * _LC


def _linearize_body(lo_ref, hi_ref, out_ref):
    out_ref[...] = jnp.concatenate([lo_ref[...].T, hi_ref[...].T], axis=1)


def _gather_body(xt_hbm, table_hbm, out_hbm, idx0, idx1, rows0, rows1, tbuf,
                 isem0, isem1, gsem0, gsem1, osem):
    wid = lax.axis_index("s") * _NC + lax.axis_index("c")
    b0 = wid * _BPW

    idxb = (idx0, idx1)
    rows = (rows0, rows1)
    isem = (isem0, isem1)
    gsem = (gsem0, gsem1)

    def fetch_idx(h, s):
        pltpu.async_copy(xt_hbm.at[h, pl.ds(b0, _BPW)], idxb[s], isem[s])

    def start_gather(h, s):
        pltpu.make_async_copy(
            xt_hbm.at[h, pl.ds(b0, _BPW)], idxb[s], isem[s]
        ).wait()
        @pl.loop(0, _BPW, step=16)
        def _(g):
            v = idxb[s][pl.ds(g, 16)]
            m = jnp.where(v >= _H, jnp.int32(1), jnp.int32(0))
            idxb[s][pl.ds(g, 16)] = 2 * (v - m * _H) + m

        pltpu.async_copy(table_hbm.at[idxb[s]], rows[s], gsem[s])

    def wait_gather(s):
        pltpu.make_async_copy(table_hbm.at[idxb[s]], rows[s], gsem[s]).wait()

    c0 = wid * (_BPW // 128)

    def wait_store(h):
        pltpu.make_async_copy(
            tbuf, out_hbm.at[h, :, pl.ds(c0, _BPW // 128), :], osem
        ).wait()

    fetch_idx(0, 0)
    fetch_idx(1, 1)
    start_gather(0, 0)

    @pl.loop(0, _HIST, step=2)
    def _(h):
        for s in range(2):
            cur = h + s
            wait_gather(s)

            @pl.when(cur + 2 < _HIST)
            def _():
                fetch_idx(cur + 2, s)

            @pl.when(cur + 1 < _HIST)
            def _():
                start_gather(cur + 1, 1 - s)

            @pl.when(cur >= 1)
            def _():
                wait_store(cur - 1)

            @pl.loop(0, _D)
            def _(d):
                iv = lax.iota(jnp.int32, 16)
                dv = (d + iv) & (_D - 1)
                rv = dv >> 3
                wb = (dv & 7) * 128 + iv
                for g0 in range(0, _BPW // 16, 8):
                    vs = [
                        plsc.load_gather(rows[s], [iv + (g0 + k) * 16, dv])
                        for k in range(8)
                    ]
                    for k in range(8):
                        g = g0 + k
                        cv = jnp.full((16,), g >> 3, jnp.int32)
                        wv = wb + (g & 7) * 16
                        plsc.store_scatter(tbuf, [rv, cv, wv], vs[k])

            pltpu.async_copy(
                tbuf, out_hbm.at[cur, :, pl.ds(c0, _BPW // 128), :], osem
            )

    wait_store(_HIST - 1)


@jax.jit
def _embedding_lookup(x_t, table_t):
    lin = pl.pallas_call(
        _linearize_body,
        grid=(_NJ,),
        in_specs=[
            pl.BlockSpec((_D, _LC), lambda j: (0, j)),
            pl.BlockSpec(
                (_D, _LC),
                lambda j: (0, jnp.where(j + _NJ < pl.cdiv(_NUM_ITEMS, _LC),
                                        j + _NJ, 0)),
            ),
        ],
        out_specs=pl.BlockSpec((_LC, 2 * _D), lambda j: (j, 0)),
        out_shape=jax.ShapeDtypeStruct((_H, 2 * _D), jnp.float32),
    )(table_t, table_t)
    lin = lin.reshape(2 * _H, _D)

    mesh = plsc.VectorSubcoreMesh(core_axis_name="c", subcore_axis_name="s")
    call = functools.partial(
        pl.kernel,
        mesh=mesh,
        out_type=jax.ShapeDtypeStruct(
            (_HIST, _D // 8, _BATCH // 128, 1024), jnp.float32
        ),
        scratch_types=(
            [pltpu.VMEM((_BPW,), jnp.int32) for _ in range(2)]
            + [pltpu.VMEM((_BPW, _D), jnp.float32) for _ in range(2)]
            + [pltpu.VMEM((_D // 8, _BPW // 128, 1024), jnp.float32)]
            + [pltpu.SemaphoreType.DMA for _ in range(5)]
        ),
        compiler_params=pltpu.CompilerParams(
            use_tc_tiling_on_sc=False, needs_layout_passes=False
        ),
    )(_gather_body)
    return call(x_t, lin)


def kernel(x, embedding_u):
    x_t = jnp.transpose(x).astype(jnp.int32)
    table_t = jnp.transpose(embedding_u)
    out4 = _embedding_lookup(x_t, table_t)
    r5 = out4.reshape(_HIST, 8, _BATCH // 128, 8, 128)
    t5 = r5.transpose(2, 4, 0, 1, 3)
    return t5.reshape(_BATCH, _HIST, _D)


# speedup vs baseline: 5.2956x; 1.1380x over previous
import functools

import jax
import jax.numpy as jnp
from jax import lax
from jax.experimental import pallas as pl
from jax.experimental.pallas import tpu as pltpu
from jax.experimental.pallas import tpu_sc as plsc

_NUM_ITEMS = 1000000
_D = 64
_BATCH = 16384
_HIST = 50

_NC = 2
_NS = 16
_NW = _NC * _NS
_BPW = _BATCH // _NW

_LC = 8192
_NJ = pl.cdiv(_NUM_ITEMS // 2, _LC)
_H = _NJ * _LC


def _linearize_body(lo_ref, hi_ref, out_ref):
    out_ref[...] = jnp.concatenate([lo_ref[...].T, hi_ref[...].T], axis=1)


_CS = _BPW // 2
_NCH = 2 * _HIST
_CT = _CS // 128


def _gather_body(xt_hbm, table_hbm, out_hbm, idx0, idx1, rows0, rows1,
                 tbuf0, tbuf1, isem0, isem1, gsem0, gsem1, osem0, osem1):
    wid = lax.axis_index("s") * _NC + lax.axis_index("c")
    b0 = wid * _BPW
    c0 = wid * (_BPW // 128)

    idxb = (idx0, idx1)
    rows = (rows0, rows1)
    tbuf = (tbuf0, tbuf1)
    isem = (isem0, isem1)
    gsem = (gsem0, gsem1)
    osem = (osem0, osem1)

    def xt_slice(c):
        return xt_hbm.at[c >> 1, pl.ds(b0 + (c & 1) * _CS, _CS)]

    def out_slice(c):
        return out_hbm.at[c >> 1, :, pl.ds(c0 + (c & 1) * _CT, _CT), :]

    def fetch_idx(c, s):
        pltpu.async_copy(xt_slice(c), idxb[s], isem[s])

    def start_gather(c, s):
        pltpu.make_async_copy(xt_slice(c), idxb[s], isem[s]).wait()
        @pl.loop(0, _CS, step=16)
        def _(g):
            v = idxb[s][pl.ds(g, 16)]
            m = jnp.where(v >= _H, jnp.int32(1), jnp.int32(0))
            idxb[s][pl.ds(g, 16)] = 2 * (v - m * _H) + m

        pltpu.async_copy(table_hbm.at[idxb[s]], rows[s], gsem[s])

    def wait_gather(s):
        pltpu.make_async_copy(table_hbm.at[idxb[s]], rows[s], gsem[s]).wait()

    def wait_store(c, s):
        pltpu.make_async_copy(tbuf[s], out_slice(c), osem[s]).wait()

    fetch_idx(0, 0)
    fetch_idx(1, 1)
    start_gather(0, 0)

    @pl.loop(0, _NCH, step=2)
    def _(cc):
        for s in range(2):
            cur = cc + s
            wait_gather(s)

            @pl.when(cur + 2 < _NCH)
            def _():
                fetch_idx(cur + 2, s)

            @pl.when(cur + 1 < _NCH)
            def _():
                start_gather(cur + 1, 1 - s)

            @pl.when(cur >= 2)
            def _():
                wait_store(cur - 2, s)

            @pl.loop(0, _D)
            def _(d):
                iv = lax.iota(jnp.int32, 16)
                dv = (d + iv) & (_D - 1)
                rv = dv >> 3
                wb = (dv & 7) * 128 + iv
                for g0 in range(0, _CS // 16, 8):
                    vs = [
                        plsc.load_gather(rows[s], [iv + (g0 + k) * 16, dv])
                        for k in range(8)
                    ]
                    for k in range(8):
                        g = g0 + k
                        cv = jnp.full((16,), g >> 3, jnp.int32)
                        wv = wb + (g & 7) * 16
                        plsc.store_scatter(tbuf[s], [rv, cv, wv], vs[k])

            pltpu.async_copy(tbuf[s], out_slice(cur), osem[s])

    wait_store(_NCH - 2, 0)
    wait_store(_NCH - 1, 1)


@jax.jit
def _embedding_lookup(x_t, table_t):
    lin = pl.pallas_call(
        _linearize_body,
        grid=(_NJ,),
        in_specs=[
            pl.BlockSpec((_D, _LC), lambda j: (0, j)),
            pl.BlockSpec(
                (_D, _LC),
                lambda j: (0, jnp.where(j + _NJ < pl.cdiv(_NUM_ITEMS, _LC),
                                        j + _NJ, 0)),
            ),
        ],
        out_specs=pl.BlockSpec((_LC, 2 * _D), lambda j: (j, 0)),
        out_shape=jax.ShapeDtypeStruct((_H, 2 * _D), jnp.float32),
    )(table_t, table_t)
    lin = lin.reshape(2 * _H, _D)

    mesh = plsc.VectorSubcoreMesh(core_axis_name="c", subcore_axis_name="s")
    call = functools.partial(
        pl.kernel,
        mesh=mesh,
        out_type=jax.ShapeDtypeStruct(
            (_HIST, _D // 8, _BATCH // 128, 1024), jnp.float32
        ),
        scratch_types=(
            [pltpu.VMEM((_CS,), jnp.int32) for _ in range(2)]
            + [pltpu.VMEM((_CS, _D), jnp.float32) for _ in range(2)]
            + [pltpu.VMEM((_D // 8, _CT, 1024), jnp.float32) for _ in range(2)]
            + [pltpu.SemaphoreType.DMA for _ in range(6)]
        ),
        compiler_params=pltpu.CompilerParams(
            use_tc_tiling_on_sc=False, needs_layout_passes=False
        ),
    )(_gather_body)
    return call(x_t, lin)


def kernel(x, embedding_u):
    x_t = jnp.transpose(x).astype(jnp.int32)
    table_t = jnp.transpose(embedding_u)
    out4 = _embedding_lookup(x_t, table_t)
    r5 = out4.reshape(_HIST, 8, _BATCH // 128, 8, 128)
    t5 = r5.transpose(2, 4, 0, 1, 3)
    return t5.reshape(_BATCH, _HIST, _D)


# speedup vs baseline: 5.4833x; 1.0355x over previous
import functools

import jax
import jax.numpy as jnp
from jax import lax
from jax.experimental import pallas as pl
from jax.experimental.pallas import tpu as pltpu
from jax.experimental.pallas import tpu_sc as plsc

_NUM_ITEMS = 1000000
_D = 64
_BATCH = 16384
_HIST = 50

_NC = 2
_NS = 16
_NW = _NC * _NS
_BPW = _BATCH // _NW

_LC = 16384
_NJ = pl.cdiv(_NUM_ITEMS // 2, _LC)
_H = _NJ * _LC


def _linearize_body(lo_ref, hi_ref, out_ref):
    out_ref[...] = jnp.concatenate([lo_ref[...].T, hi_ref[...].T], axis=1)


_CS = _BPW // 2
_NCH = 2 * _HIST
_CT = _CS // 128


def _gather_body(xt_hbm, table_hbm, out_hbm, idx0, idx1, rows0, rows1,
                 tbuf0, tbuf1, isem0, isem1, gsem0, gsem1, osem0, osem1):
    wid = lax.axis_index("s") * _NC + lax.axis_index("c")
    b0 = wid * _BPW
    c0 = wid * (_BPW // 128)

    idxb = (idx0, idx1)
    rows = (rows0, rows1)
    tbuf = (tbuf0, tbuf1)
    isem = (isem0, isem1)
    gsem = (gsem0, gsem1)
    osem = (osem0, osem1)

    def xt_slice(c):
        return xt_hbm.at[c >> 1, pl.ds(b0 + (c & 1) * _CS, _CS)]

    def out_slice(c):
        return out_hbm.at[c >> 1, :, pl.ds(c0 + (c & 1) * _CT, _CT), :]

    def fetch_idx(c, s):
        pltpu.async_copy(xt_slice(c), idxb[s], isem[s])

    def start_gather(c, s):
        pltpu.make_async_copy(xt_slice(c), idxb[s], isem[s]).wait()
        @pl.loop(0, _CS, step=16)
        def _(g):
            v = idxb[s][pl.ds(g, 16)]
            m = jnp.where(v >= _H, jnp.int32(1), jnp.int32(0))
            idxb[s][pl.ds(g, 16)] = 2 * (v - m * _H) + m

        pltpu.async_copy(table_hbm.at[idxb[s]], rows[s], gsem[s])

    def wait_gather(s):
        pltpu.make_async_copy(table_hbm.at[idxb[s]], rows[s], gsem[s]).wait()

    def wait_store(c, s):
        pltpu.make_async_copy(tbuf[s], out_slice(c), osem[s]).wait()

    fetch_idx(0, 0)
    fetch_idx(1, 1)
    start_gather(0, 0)

    @pl.loop(0, _NCH, step=2)
    def _(cc):
        for s in range(2):
            cur = cc + s
            wait_gather(s)

            @pl.when(cur + 2 < _NCH)
            def _():
                fetch_idx(cur + 2, s)

            @pl.when(cur + 1 < _NCH)
            def _():
                start_gather(cur + 1, 1 - s)

            @pl.when(cur >= 2)
            def _():
                wait_store(cur - 2, s)

            @pl.loop(0, _D)
            def _(d):
                iv = lax.iota(jnp.int32, 16)
                dv = (d + iv) & (_D - 1)
                rv = dv >> 3
                wb = (dv & 7) * 128 + iv
                for g0 in range(0, _CS // 16, 8):
                    vs = [
                        plsc.load_gather(rows[s], [iv + (g0 + k) * 16, dv])
                        for k in range(8)
                    ]
                    for k in range(8):
                        g = g0 + k
                        cv = jnp.full((16,), g >> 3, jnp.int32)
                        wv = wb + (g & 7) * 16
                        plsc.store_scatter(tbuf[s], [rv, cv, wv], vs[k])

            pltpu.async_copy(tbuf[s], out_slice(cur), osem[s])

    wait_store(_NCH - 2, 0)
    wait_store(_NCH - 1, 1)


@jax.jit
def _embedding_lookup(x_t, table_t):
    lin = pl.pallas_call(
        _linearize_body,
        grid=(_NJ,),
        in_specs=[
            pl.BlockSpec((_D, _LC), lambda j: (0, j)),
            pl.BlockSpec(
                (_D, _LC),
                lambda j: (0, jnp.where(j + _NJ < pl.cdiv(_NUM_ITEMS, _LC),
                                        j + _NJ, 0)),
            ),
        ],
        out_specs=pl.BlockSpec((_LC, 2 * _D), lambda j: (j, 0)),
        out_shape=jax.ShapeDtypeStruct((_H, 2 * _D), jnp.float32),
    )(table_t, table_t)
    lin = lin.reshape(2 * _H, _D)

    mesh = plsc.VectorSubcoreMesh(core_axis_name="c", subcore_axis_name="s")
    call = functools.partial(
        pl.kernel,
        mesh=mesh,
        out_type=jax.ShapeDtypeStruct(
            (_HIST, _D // 8, _BATCH // 128, 1024), jnp.float32
        ),
        scratch_types=(
            [pltpu.VMEM((_CS,), jnp.int32) for _ in range(2)]
            + [pltpu.VMEM((_CS, _D), jnp.float32) for _ in range(2)]
            + [pltpu.VMEM((_D // 8, _CT, 1024), jnp.float32) for _ in range(2)]
            + [pltpu.SemaphoreType.DMA for _ in range(6)]
        ),
        compiler_params=pltpu.CompilerParams(
            use_tc_tiling_on_sc=False, needs_layout_passes=False
        ),
    )(_gather_body)
    return call(x_t, lin)


def kernel(x, embedding_u):
    x_t = jnp.transpose(x).astype(jnp.int32)
    table_t = jnp.transpose(embedding_u)
    out4 = _embedding_lookup(x_t, table_t)
    r5 = out4.reshape(_HIST, 8, _BATCH // 128, 8, 128)
    t5 = r5.transpose(2, 4, 0, 1, 3)
    return t5.reshape(_BATCH, _HIST, _D)
